# baseline (speedup 1.0000x reference)
import functools
import jax
import jax.numpy as jnp
from jax import lax
from jax.experimental import pallas as pl
from jax.experimental.pallas import tpu as pltpu
from jax.experimental.pallas import tpu_sc as plsc

DEPTH_ = 1000
NC_ = 2
NS_ = 16
LANES_ = 16
NW_ = NC_ * NS_
N_TOK_ = 51200
TOK_PER_W_ = N_TOK_ // NW_
CHUNK_ = 80
NCHUNK_ = TOK_PER_W_ // CHUNK_
GROUPS_ = CHUNK_ // LANES_


def _sc_body(inp_hbm, out_hbm, idx_v, buf, sem):
    wid = lax.axis_index("s") * NC_ + lax.axis_index("c")
    base = wid * TOK_PER_W_

    pltpu.sync_copy(inp_hbm.at[pl.ds(base, TOK_PER_W_)], idx_v)

    zeros16 = jnp.zeros((LANES_,), jnp.float32)
    ones16 = jnp.ones((LANES_,), jnp.float32)
    lane = lax.iota(jnp.int32, LANES_)

    def zfill(j, _):
        for jj in range(10):
            buf[pl.ds(j * 160 + jj * LANES_, LANES_)] = zeros16
        return 0

    lax.fori_loop(0, CHUNK_ * DEPTH_ // 160, zfill, 0)

    def stamp(c, val16):
        for g in range(GROUPS_):
            vals = idx_v[pl.ds(c * CHUNK_ + g * LANES_, LANES_)]
            offs = (lane + g * LANES_) * DEPTH_ + vals
            plsc.store_scatter(buf, [offs], val16, mask=vals != 0)

    def chunk_body(c, _):
        stamp(c, ones16)
        pltpu.sync_copy(
            buf, out_hbm.at[pl.ds((base + c * CHUNK_) * DEPTH_, CHUNK_ * DEPTH_)]
        )
        stamp(c, zeros16)
        return 0

    lax.fori_loop(0, NCHUNK_, chunk_body, 0)


def kernel(inputs):
    b, t = inputs.shape
    flat = inputs.reshape(b * t)
    mesh = plsc.VectorSubcoreMesh(core_axis_name="c", subcore_axis_name="s")
    k = functools.partial(
        pl.kernel,
        mesh=mesh,
        out_type=jax.ShapeDtypeStruct((b * t * DEPTH_,), jnp.float32),
        scratch_types=[
            pltpu.VMEM((TOK_PER_W_,), jnp.int32),
            pltpu.VMEM((CHUNK_ * DEPTH_,), jnp.float32),
            pltpu.SemaphoreType.DMA,
        ],
        compiler_params=pltpu.CompilerParams(needs_layout_passes=False),
    )(_sc_body)
    out = k(flat)
    return out.reshape(b, t, DEPTH_)


# speedup vs baseline: 1.7995x; 1.7995x over previous
import functools
import jax
import jax.numpy as jnp
from jax import lax
from jax.experimental import pallas as pl
from jax.experimental.pallas import tpu as pltpu
from jax.experimental.pallas import tpu_sc as plsc

B_ = 1024
T_ = 50
DEPTH_ = 1000
NC_ = 2
NS_ = 16
LANES_ = 16
NW_ = NC_ * NS_
ROWS_PER_W_ = B_ // NW_
TOK_PER_W_ = ROWS_PER_W_ * T_
GROUPS_ = (T_ + LANES_ - 1) // LANES_
IDX_PAD_ = 64


def _sc_body(inp_hbm, zeros_hbm, out_hbm, idx_v, buf_a, buf_b, sem_a, sem_b):
    wid = lax.axis_index("s") * NC_ + lax.axis_index("c")
    tok_base = wid * TOK_PER_W_
    row_base = wid * ROWS_PER_W_

    pltpu.sync_copy(
        inp_hbm.at[pl.ds(tok_base, TOK_PER_W_)], idx_v.at[pl.ds(0, TOK_PER_W_)]
    )
    pltpu.sync_copy(zeros_hbm, buf_a)
    pltpu.sync_copy(zeros_hbm, buf_b)

    zeros16 = jnp.zeros((LANES_,), jnp.float32)
    ones16 = jnp.ones((LANES_,), jnp.float32)
    lane = lax.iota(jnp.int32, LANES_)

    def stamp(r_local, buf2d, val16):
        for g in range(GROUPS_):
            vals = idx_v[pl.ds(r_local * T_ + g * LANES_, LANES_)]
            rows = lane + g * LANES_
            live = vals != 0
            if (g + 1) * LANES_ > T_:
                live = live & (rows < T_)
            plsc.store_scatter(buf2d, [rows, vals], val16, mask=live)

    bufs = (buf_a, buf_b)
    sems = (sem_a, sem_b)

    def pair_body(p, _):
        for half in range(2):
            r = p * 2 + half
            buf = bufs[half]
            sem = sems[half]

            @pl.when(p > 0)
            def _():
                pltpu.make_async_copy(buf, out_hbm.at[row_base + r - 2], sem).wait()
                stamp(r - 2, buf, zeros16)

            stamp(r, buf, ones16)
            pltpu.make_async_copy(buf, out_hbm.at[row_base + r], sem).start()
        return 0

    lax.fori_loop(0, ROWS_PER_W_ // 2, pair_body, 0)

    for half in range(2):
        r = ROWS_PER_W_ - 2 + half
        pltpu.make_async_copy(bufs[half], out_hbm.at[row_base + r], sems[half]).wait()


def kernel(inputs):
    b, t = inputs.shape
    flat = inputs.reshape(b * t)
    zeros2d = jnp.zeros((t, DEPTH_), jnp.float32)
    mesh = plsc.VectorSubcoreMesh(core_axis_name="c", subcore_axis_name="s")
    k = functools.partial(
        pl.kernel,
        mesh=mesh,
        out_type=jax.ShapeDtypeStruct((b, t, DEPTH_), jnp.float32),
        scratch_types=[
            pltpu.VMEM((TOK_PER_W_ + IDX_PAD_,), jnp.int32),
            pltpu.VMEM((T_, DEPTH_), jnp.float32),
            pltpu.VMEM((T_, DEPTH_), jnp.float32),
            pltpu.SemaphoreType.DMA,
            pltpu.SemaphoreType.DMA,
        ],
        compiler_params=pltpu.CompilerParams(needs_layout_passes=False),
    )(_sc_body)
    return k(flat, zeros2d)


# speedup vs baseline: 1.8033x; 1.0021x over previous
import functools
import jax
import jax.numpy as jnp
from jax import lax
from jax.experimental import pallas as pl
from jax.experimental.pallas import tpu as pltpu
from jax.experimental.pallas import tpu_sc as plsc

B_ = 1024
T_ = 50
DEPTH_ = 1000
NC_ = 2
NS_ = 16
LANES_ = 16
NW_ = NC_ * NS_
ROWS_PER_W_ = B_ // NW_
TOK_PER_W_ = ROWS_PER_W_ * T_
GROUPS_ = (T_ + LANES_ - 1) // LANES_
IDX_PAD_ = 64


def _sc_body(inp_hbm, zeros_hbm, out_hbm, idx_v, buf_a, buf_b, sem_a, sem_b):
    wid = lax.axis_index("s") * NC_ + lax.axis_index("c")
    tok_base = wid * TOK_PER_W_
    row_base = wid * ROWS_PER_W_

    pltpu.sync_copy(
        inp_hbm.at[pl.ds(tok_base, TOK_PER_W_)], idx_v.at[pl.ds(0, TOK_PER_W_)]
    )
    pltpu.sync_copy(zeros_hbm, buf_a)
    pltpu.sync_copy(zeros_hbm, buf_b)

    zeros16 = jnp.zeros((LANES_,), jnp.float32)
    ones16 = jnp.ones((LANES_,), jnp.float32)
    lane = lax.iota(jnp.int32, LANES_)

    def stamp(r_local, buf2d, val16):
        for g in range(GROUPS_):
            vals = idx_v[pl.ds(r_local * T_ + g * LANES_, LANES_)]
            rows = lane + g * LANES_
            live = vals != 0
            if (g + 1) * LANES_ > T_:
                live = live & (rows < T_)
            plsc.store_scatter(buf2d, [rows, vals], val16, mask=live)

    bufs = (buf_a, buf_b)
    sems = (sem_a, sem_b)

    def pair_body(p, _):
        for half in range(2):
            r = p * 2 + half
            buf = bufs[half]
            sem = sems[half]

            @pl.when(p > 0)
            def _():
                pltpu.make_async_copy(buf, out_hbm.at[row_base + r - 2], sem).wait()
                stamp(r - 2, buf, zeros16)

            stamp(r, buf, ones16)
            pltpu.make_async_copy(buf, out_hbm.at[row_base + r], sem).start()
        return 0

    lax.fori_loop(0, ROWS_PER_W_ // 2, pair_body, 0)

    for half in range(2):
        r = ROWS_PER_W_ - 2 + half
        pltpu.make_async_copy(bufs[half], out_hbm.at[row_base + r], sems[half]).wait()


def kernel(inputs):
    b, t = inputs.shape
    flat = inputs.reshape(b * t)
    zeros2d = jnp.zeros((t, DEPTH_), jnp.float32)
    mesh = plsc.VectorSubcoreMesh(core_axis_name="c", subcore_axis_name="s")
    k = functools.partial(
        pl.kernel,
        mesh=mesh,
        out_type=jax.ShapeDtypeStruct((b, t, DEPTH_), jnp.float32),
        scratch_types=[
            pltpu.VMEM((TOK_PER_W_ + IDX_PAD_,), jnp.int32),
            pltpu.VMEM((T_, DEPTH_), jnp.float32),
            pltpu.VMEM((T_, DEPTH_), jnp.float32),
            pltpu.SemaphoreType.DMA,
            pltpu.SemaphoreType.DMA,
        ],
        compiler_params=pltpu.CompilerParams(
            needs_layout_passes=False, use_tc_tiling_on_sc=True
        ),
    )(_sc_body)
    return k(flat, zeros2d)


# speedup vs baseline: 1.8549x; 1.0286x over previous
import functools
import jax
import jax.numpy as jnp
from jax import lax
from jax.experimental import pallas as pl
from jax.experimental.pallas import tpu as pltpu
from jax.experimental.pallas import tpu_sc as plsc

B_ = 1024
T_ = 50
DEPTH_ = 1000
NC_ = 2
NS_ = 16
LANES_ = 16
NW_ = NC_ * NS_
ROWS_PER_W_ = B_ // NW_
TOK_PER_W_ = ROWS_PER_W_ * T_
RB_ = 2
CTOK_ = RB_ * T_
NCHUNK_ = ROWS_PER_W_ // RB_
GROUPS_ = (CTOK_ + LANES_ - 1) // LANES_
IDX_PAD_ = 64


def _sc_body(inp_hbm, zeros_hbm, out_hbm, idx_v, buf, sem):
    wid = lax.axis_index("s") * NC_ + lax.axis_index("c")
    tok_base = wid * TOK_PER_W_
    row_base = wid * ROWS_PER_W_

    pltpu.sync_copy(
        inp_hbm.at[pl.ds(tok_base, TOK_PER_W_)], idx_v.at[pl.ds(0, TOK_PER_W_)]
    )
    pltpu.sync_copy(zeros_hbm, buf)

    zeros16 = jnp.zeros((LANES_,), jnp.float32)
    ones16 = jnp.ones((LANES_,), jnp.float32)
    lane = lax.iota(jnp.int32, LANES_)

    def stamp(c, val16):
        for g in range(GROUPS_):
            tok = lane + g * LANES_
            vals = idx_v[pl.ds(c * CTOK_ + g * LANES_, LANES_)]
            row = tok // T_
            tcol = tok - row * T_
            live = vals != 0
            if (g + 1) * LANES_ > CTOK_:
                live = live & (tok < CTOK_)
            plsc.store_scatter(buf, [row, tcol, vals], val16, mask=live)

    def chunk_body(c, _):
        stamp(c, ones16)
        pltpu.sync_copy(buf, out_hbm.at[pl.ds(row_base + c * RB_, RB_)])
        stamp(c, zeros16)
        return 0

    lax.fori_loop(0, NCHUNK_, chunk_body, 0)


def kernel(inputs):
    b, t = inputs.shape
    flat = inputs.reshape(b * t)
    zeros3d = jnp.zeros((RB_, t, DEPTH_), jnp.float32)
    mesh = plsc.VectorSubcoreMesh(core_axis_name="c", subcore_axis_name="s")
    k = functools.partial(
        pl.kernel,
        mesh=mesh,
        out_type=jax.ShapeDtypeStruct((b, t, DEPTH_), jnp.float32),
        scratch_types=[
            pltpu.VMEM((TOK_PER_W_ + IDX_PAD_,), jnp.int32),
            pltpu.VMEM((RB_, T_, DEPTH_), jnp.float32),
            pltpu.SemaphoreType.DMA,
        ],
        compiler_params=pltpu.CompilerParams(needs_layout_passes=False),
    )(_sc_body)
    return k(flat, zeros3d)
